# baseline (speedup 1.0000x reference)
import jax
import jax.numpy as jnp
from jax.experimental import pallas as pl
from jax.experimental.pallas import tpu as pltpu

CHANS, SEQ, FEAT = 8, 4096, 1024
N_ROWS = SEQ // 4
ROWS = CHANS * SEQ
BLOCK_S = 2048
SEQ_BLOCKS = SEQ // BLOCK_S
CHUNK = N_ROWS // CHANS


def _row_scale_body(idx_ref, warp_ref, x_ref, out_ref, cur_ref, nxt_ref):
    s = pl.program_id(0)
    c = pl.program_id(1)

    @pl.when((s == 0) & (c == 0))
    def _build_first_half_scale():
        rows = jax.lax.broadcasted_iota(jnp.int32, (BLOCK_S, 1), 0)
        eq = rows == idx_ref[...]
        contrib = jnp.where(eq, warp_ref[...] - 1.0, 0.0)
        cur_ref[...] = 1.0 + jnp.sum(contrib, axis=1, keepdims=True)

    @pl.when((s > 0) & (c == 0))
    def _advance_scale():
        cur_ref[...] = nxt_ref[...]

    @pl.when(s < SEQ_BLOCKS - 1)
    def _accumulate_next_half_scale():
        rows = jax.lax.broadcasted_iota(jnp.int32, (BLOCK_S, 1), 0) + (s + 1) * BLOCK_S
        idx_chunk = idx_ref[0, pl.ds(c * CHUNK, CHUNK)][None, :]
        w_chunk = warp_ref[0, pl.ds(c * CHUNK, CHUNK)][None, :]
        eq = rows == idx_chunk
        contrib = jnp.sum(jnp.where(eq, w_chunk - 1.0, 0.0), axis=1, keepdims=True)
        base = jnp.where(c == 0, 1.0, 0.0)
        nxt_ref[...] = jnp.where(c == 0, base + contrib, nxt_ref[...] + contrib)

    out_ref[...] = x_ref[...] * cur_ref[...]


def kernel(x, idxs, warp):
    idxs2d = idxs.reshape(1, N_ROWS)
    warp2d = warp.reshape(1, N_ROWS)
    x2d = x.reshape(ROWS, FEAT)
    out2d = pl.pallas_call(
        _row_scale_body,
        grid=(SEQ_BLOCKS, CHANS),
        in_specs=[
            pl.BlockSpec((1, N_ROWS), lambda s, c: (0, 0)),
            pl.BlockSpec((1, N_ROWS), lambda s, c: (0, 0)),
            pl.BlockSpec((BLOCK_S, FEAT), lambda s, c: (c * SEQ_BLOCKS + s, 0)),
        ],
        out_specs=pl.BlockSpec((BLOCK_S, FEAT), lambda s, c: (c * SEQ_BLOCKS + s, 0)),
        out_shape=jax.ShapeDtypeStruct((ROWS, FEAT), x.dtype),
        scratch_shapes=[
            pltpu.VMEM((BLOCK_S, 1), jnp.float32),
            pltpu.VMEM((BLOCK_S, 1), jnp.float32),
        ],
        compiler_params=pltpu.CompilerParams(
            dimension_semantics=("arbitrary", "arbitrary"),
        ),
    )(idxs2d, warp2d, x2d)
    return out2d.reshape(CHANS, SEQ, FEAT)


# speedup vs baseline: 1.0031x; 1.0031x over previous
import jax
import jax.numpy as jnp
from jax.experimental import pallas as pl
from jax.experimental.pallas import tpu as pltpu

CHANS, SEQ, FEAT = 8, 4096, 1024
N_ROWS = SEQ // 4
BLOCK_S = 2048
SEQ_BLOCKS = SEQ // BLOCK_S


def _row_scale_body(idx_ref, warp_ref, x_ref, out_ref, scale_ref):
    c = pl.program_id(1)

    @pl.when(c == 0)
    def _compute_scale():
        s = pl.program_id(0)
        rows = jax.lax.broadcasted_iota(jnp.int32, (BLOCK_S, 1), 0) + s * BLOCK_S
        eq = rows == idx_ref[...]
        contrib = jnp.where(eq, warp_ref[...] - 1.0, 0.0)
        scale_ref[...] = 1.0 + jnp.sum(contrib, axis=1, keepdims=True)

    out_ref[...] = x_ref[...] * scale_ref[...][None, :, :]


def kernel(x, idxs, warp):
    idxs2d = idxs.reshape(1, N_ROWS)
    warp2d = warp.reshape(1, N_ROWS)
    return pl.pallas_call(
        _row_scale_body,
        grid=(SEQ_BLOCKS, CHANS),
        in_specs=[
            pl.BlockSpec((1, N_ROWS), lambda s, c: (0, 0)),
            pl.BlockSpec((1, N_ROWS), lambda s, c: (0, 0)),
            pl.BlockSpec((1, BLOCK_S, FEAT), lambda s, c: (c, s, 0)),
        ],
        out_specs=pl.BlockSpec((1, BLOCK_S, FEAT), lambda s, c: (c, s, 0)),
        out_shape=jax.ShapeDtypeStruct((CHANS, SEQ, FEAT), x.dtype),
        scratch_shapes=[pltpu.VMEM((BLOCK_S, 1), jnp.float32)],
        compiler_params=pltpu.CompilerParams(
            dimension_semantics=("arbitrary", "arbitrary"),
        ),
    )(idxs2d, warp2d, x)
